# baseline (speedup 1.0000x reference)
import numpy as np
import jax
import jax.numpy as jnp
from jax.experimental import pallas as pl

PERIOD = 144
TREND_K = 25


def _build_mats(t):
    idx = np.arange(t)
    a = (np.abs(idx[:, None] - idx[None, :]) <= TREND_K // 2).astype(np.float32) / TREND_K
    num_full = t // PERIOD
    d = (idx[:, None] % PERIOD == idx[None, :] % PERIOD).astype(np.float32) / num_full
    h = np.eye(t, dtype=np.float32) - 0.5 * a - 0.5 * d
    return a, d, h


def _body(x_ref, a_ref, d_ref, h_ref, hf_ref, daily_ref, trend_ref):
    xb = x_ref[...]
    prec = jax.lax.Precision.HIGHEST
    trend_ref[...] = jnp.dot(xb, a_ref[...], preferred_element_type=jnp.float32,
                             precision=prec)
    daily_ref[...] = jnp.dot(xb, d_ref[...], preferred_element_type=jnp.float32,
                             precision=prec)
    hf_ref[...] = jnp.dot(xb, h_ref[...], preferred_element_type=jnp.float32,
                          precision=prec)


def kernel(x):
    b, c, n, t = x.shape
    r = b * c * n
    xf = x.reshape(r, t)
    a, d, h = _build_mats(t)
    a, d, h = jnp.asarray(a), jnp.asarray(d), jnp.asarray(h)

    br = 1664
    assert r % br == 0
    grid = (r // br,)
    row_spec = pl.BlockSpec((br, t), lambda i: (i, 0))
    mat_spec = pl.BlockSpec((t, t), lambda i: (0, 0))
    out = jax.ShapeDtypeStruct((r, t), jnp.float32)

    hf, daily, trend = pl.pallas_call(
        _body,
        grid=grid,
        in_specs=[row_spec, mat_spec, mat_spec, mat_spec],
        out_specs=[row_spec, row_spec, row_spec],
        out_shape=[out, out, out],
    )(xf, a, d, h)

    shape = (b, c, n, t)
    return hf.reshape(shape), daily.reshape(shape), trend.reshape(shape)
